# baseline (speedup 1.0000x reference)
import functools

import jax
import jax.numpy as jnp
from jax import lax
from jax.experimental import pallas as pl
from jax.experimental.pallas import tpu as pltpu
from jax.experimental.pallas import tpu_sc as plsc

N = 10000
E = 320000
H = 128
C = 40
NC = 2
NS = 16
NW = NC * NS
BM = 400


EPW = E // NW
CPW = EPW // C
SBC = 50
SBE = SBC * C
NSB = CPW // SBC
DEN = N + 16


def _build_sc_gat():
    mesh = plsc.VectorSubcoreMesh(core_axis_name="c", subcore_axis_name="s")
    CH = C // 2

    @functools.partial(
        pl.kernel,
        mesh=mesh,
        out_type=[
            jax.ShapeDtypeStruct((NC, N, H), jnp.float32),
            jax.ShapeDtypeStruct((NC, NS, DEN), jnp.float32),
        ],
        scratch_types=[
            pltpu.VMEM((C, H), jnp.float32),
            pltpu.VMEM((C, H), jnp.float32),
            pltpu.VMEM((C, H), jnp.float32),
            pltpu.VMEM((C, H), jnp.float32),
            pltpu.VMEM((CH, H), jnp.float32),
            pltpu.VMEM((CH, H), jnp.float32),
            pltpu.VMEM((CH,), jnp.int32),
            pltpu.VMEM((CH,), jnp.int32),
            pltpu.VMEM((C * 16,), jnp.float32),
            pltpu.VMEM((SBE + 16,), jnp.int32),
            pltpu.VMEM((SBE + 16,), jnp.int32),
            pltpu.VMEM((SBE * 4 + 16,), jnp.float32),
            pltpu.VMEM((4, H), jnp.float32),
            pltpu.VMEM((H,), jnp.float32),
            pltpu.VMEM((DEN,), jnp.float32),
            pltpu.VMEM_SHARED((N, H), jnp.float32),
            pltpu.SemaphoreType.DMA,
            pltpu.SemaphoreType.DMA,
            pltpu.SemaphoreType.DMA,
            pltpu.SemaphoreType.DMA,
            pltpu.SemaphoreType.DMA,
            pltpu.SemaphoreType.DMA,
        ],
    )
    def sc_gat(xl_hbm, xr_hbm, src_hbm, dst_hbm, ea_hbm, we_hbm, att_hbm,
               out_hbm, den_hbm, xs0, xs1, xd0, xd1, mr0, mr1, dstsc0,
               dstsc1, pb2, srcbf, dstbf, eab, wev, attv, den_local, acc,
               sems0, sems1, semd0, semd1, semsc0, semsc1):
        cid = lax.axis_index("c")
        sid = lax.axis_index("s")
        wid = sid * NC + cid
        xsb = (xs0, xs1)
        xdb = (xd0, xd1)
        gsem = ((sems0, semd0), (sems1, semd1))
        mrh = (mr0, mr1)
        dsch = (dstsc0, dstsc1)
        scsem = (semsc0, semsc1)

        pltpu.sync_copy(we_hbm, wev)
        pltpu.sync_copy(att_hbm, attv)

        z16 = jnp.zeros((16,), jnp.float32)

        def zrow(r, carry):
            for j in range(H // 16):
                xs0[r, pl.ds(16 * j, 16)] = z16
            return carry

        lax.fori_loop(0, C, zrow, 0)

        def zden(r, carry):
            den_local[pl.ds(16 * r, 16)] = z16
            return carry

        lax.fori_loop(0, DEN // 16, zden, 0)

        row0 = sid * 632
        nfull = jnp.where(sid < NS - 1, 15, 13)

        def zacc(kk, carry):
            pltpu.sync_copy(xs0, acc.at[pl.ds(row0 + kk * C, C)])
            return carry

        lax.fori_loop(0, nfull, zacc, 0)

        @pl.when(sid < NS - 1)
        def _():
            pltpu.sync_copy(xs0.at[pl.ds(0, 32)],
                            acc.at[pl.ds(row0 + 600, 32)])

        plsc.subcore_barrier()

        wej = [[wev[d, pl.ds(16 * j, 16)] for j in range(8)] for d in range(4)]
        attj = [attv[pl.ds(16 * j, 16)] for j in range(8)]
        lane = lax.iota(jnp.int32, 16)
        bfly = [lane ^ sh for sh in (8, 4, 2, 1)]
        lane0 = lane == 0
        e0 = wid * EPW

        def issue(c, b):
            pltpu.async_copy(xl_hbm.at[srcbf.at[pl.ds(c * C, C)]],
                             xsb[b], gsem[b][0])
            pltpu.async_copy(xr_hbm.at[dstbf.at[pl.ds(c * C, C)]],
                             xdb[b], gsem[b][1])

        def wait_g(b):
            pltpu.make_async_copy(xl_hbm.at[pl.ds(0, C)], xsb[b],
                                  gsem[b][0]).wait()
            pltpu.make_async_copy(xr_hbm.at[pl.ds(0, C)], xdb[b],
                                  gsem[b][1]).wait()

        def wait_sc(hh):
            pltpu.make_async_copy(mrh[hh], acc.at[dsch[hh]],
                                  scsem[hh]).wait()

        def sb_body(sb, carry):
            sbe0 = e0 + sb * SBE
            pltpu.sync_copy(src_hbm.at[pl.ds(sbe0, SBE)],
                            srcbf.at[pl.ds(0, SBE)])
            pltpu.sync_copy(dst_hbm.at[pl.ds(sbe0, SBE)],
                            dstbf.at[pl.ds(0, SBE)])
            pltpu.sync_copy(ea_hbm.at[pl.ds(sbe0 * 4, SBE * 4)],
                            eab.at[pl.ds(0, SBE * 4)])
            issue(0, 0)
            issue(1, 1)

            def half(c, b, hh, first_sb):
                xs = xsb[b]
                xd = xdb[b]
                mr = mrh[hh]
                ofs = CH * hh

                @pl.when(jnp.logical_or(c >= 1, first_sb == 0))
                def _():
                    wait_sc(hh)

                def edge_body(e, ecarry):
                    for u in range(4):
                        k = ofs + 4 * e + u
                        eav = eab[pl.ds((c * C + k) * 4, 16)]
                        terms = []
                        for j in range(8):
                            q = (xs[k, pl.ds(16 * j, 16)]
                                 + xd[k, pl.ds(16 * j, 16)]
                                 + eav[0] * wej[0][j] + eav[1] * wej[1][j]
                                 + eav[2] * wej[2][j] + eav[3] * wej[3][j])
                            lr = jnp.maximum(q, 0.2 * q)
                            terms.append(attj[j] * lr)
                        while len(terms) > 1:
                            terms = [a + b for a, b in
                                     zip(terms[::2], terms[1::2])]
                        lacc = terms[0]
                        for bidx in bfly:
                            lacc = lacc + lax.gather(
                                lacc, bidx[:, None],
                                lax.GatherDimensionNumbers(
                                    offset_dims=(), collapsed_slice_dims=(0,),
                                    start_index_map=(0,)),
                                slice_sizes=(1,),
                                mode=lax.GatherScatterMode.PROMISE_IN_BOUNDS)
                        p = jnp.exp(lacc)
                        for j in range(8):
                            mr[4 * e + u, pl.ds(16 * j, 16)] = (
                                p * xs[k, pl.ds(16 * j, 16)])
                        pb2[pl.ds(16 * k, 16)] = p
                    return ecarry

                lax.fori_loop(0, CH // 4, edge_body, 0)
                dsc = dsch[hh]
                dsc[pl.ds(0, 16)] = dstbf[pl.ds(c * C + ofs, 16)]
                dsc[pl.ds(4, 16)] = dstbf[pl.ds(c * C + ofs + 4, 16)]
                pltpu.async_copy(mr, acc.at[dsc], scsem[hh], add=True)

                def den_body(e, ecarry):
                    for u in range(2):
                        k = ofs + 2 * e + u
                        dw = dstbf[pl.ds(c * C + k, 16)]
                        d_e = dw[0]
                        pw = pb2[pl.ds(16 * k, 16)]
                        w = den_local[pl.ds(d_e, 16)]
                        den_local[pl.ds(d_e, 16)] = (
                            w + jnp.where(lane0, pw, 0.0))
                    return ecarry

                lax.fori_loop(0, CH // 2, den_body, 0)

            def do_chunk(c, b, first_sb):
                wait_g(b)
                half(c, b, 0, first_sb)
                half(c, b, 1, first_sb)

                @pl.when(c + 2 < SBC)
                def _():
                    issue(c + 2, b)

            def pair_body(tt, carry2):
                do_chunk(2 * tt, 0, carry2)
                do_chunk(2 * tt + 1, 1, carry2)
                return carry2

            first = jnp.where(sb == 0, 1, 0)
            lax.fori_loop(0, SBC // 2, pair_body, first)
            return carry

        lax.fori_loop(0, NSB, sb_body, 0)
        wait_sc(0)
        wait_sc(1)
        pltpu.sync_copy(den_local, den_hbm.at[cid, sid])
        plsc.subcore_barrier()

        @pl.when(sid < NS - 1)
        def _():
            pltpu.sync_copy(acc.at[pl.ds(row0, 632)],
                            out_hbm.at[cid, pl.ds(row0, 632)])

        @pl.when(sid == NS - 1)
        def _():
            pltpu.sync_copy(acc.at[pl.ds(row0, 520)],
                            out_hbm.at[cid, pl.ds(row0, 520)])

    return sc_gat


_SC_GAT = _build_sc_gat()


def _mm2_body(x_ref, wl_ref, wr_ref, o1_ref, o2_ref):
    xb = x_ref[...]
    o1_ref[...] = jnp.dot(xb, wl_ref[...], preferred_element_type=jnp.float32)
    o2_ref[...] = jnp.dot(xb, wr_ref[...], preferred_element_type=jnp.float32)


def _mm2(x, wl, wr):
    m, k = x.shape
    n = wl.shape[1]
    return pl.pallas_call(
        _mm2_body,
        grid=(m // BM,),
        in_specs=[
            pl.BlockSpec((BM, k), lambda i: (i, 0)),
            pl.BlockSpec((k, n), lambda i: (0, 0)),
            pl.BlockSpec((k, n), lambda i: (0, 0)),
        ],
        out_specs=[
            pl.BlockSpec((BM, n), lambda i: (i, 0)),
            pl.BlockSpec((BM, n), lambda i: (i, 0)),
        ],
        out_shape=[
            jax.ShapeDtypeStruct((m, n), jnp.float32),
            jax.ShapeDtypeStruct((m, n), jnp.float32),
        ],
    )(x, wl, wr)


def _dred_body(d_ref, o_ref):
    o_ref[...] = jnp.sum(d_ref[...], axis=0, keepdims=True)


def _dred(den):
    d2 = den.reshape(NC * NS, DEN)
    out = pl.pallas_call(
        _dred_body,
        grid=(1,),
        in_specs=[pl.BlockSpec((NC * NS, DEN), lambda i: (0, 0))],
        out_specs=pl.BlockSpec((1, DEN), lambda i: (0, 0)),
        out_shape=jax.ShapeDtypeStruct((1, DEN), jnp.float32),
    )(d2)
    return out[:, :N].reshape(N, 1)


def _post_body(msg_ref, den_ref, b_ref, y_ref, st_ref):
    i = pl.program_id(0)
    num = msg_ref[0] + msg_ref[1]
    den = den_ref[...]
    y = jnp.maximum(num / (den + 1e-16) + b_ref[...], 0.0)
    y_ref[...] = y
    s0 = jnp.sum(y, axis=0, keepdims=True)
    s1 = jnp.sum(y * y, axis=0, keepdims=True)
    st = jnp.concatenate([s0, s1], axis=0)

    @pl.when(i == 0)
    def _():
        st_ref[...] = st

    @pl.when(i > 0)
    def _():
        st_ref[...] = st_ref[...] + st


def _post(msg, den2d, b):
    return pl.pallas_call(
        _post_body,
        grid=(N // BM,),
        in_specs=[
            pl.BlockSpec((NC, BM, H), lambda i: (0, i, 0)),
            pl.BlockSpec((BM, 1), lambda i: (i, 0)),
            pl.BlockSpec((1, H), lambda i: (0, 0)),
        ],
        out_specs=[
            pl.BlockSpec((BM, H), lambda i: (i, 0)),
            pl.BlockSpec((2, H), lambda i: (0, 0)),
        ],
        out_shape=[
            jax.ShapeDtypeStruct((N, H), jnp.float32),
            jax.ShapeDtypeStruct((2, H), jnp.float32),
        ],
    )(msg, den2d, b)


def _affine(st_ref, g_ref, be_ref):
    mean = st_ref[0:1, :] * (1.0 / N)
    var = st_ref[1:2, :] * (1.0 / N) - mean * mean
    s = g_ref[...] * lax.rsqrt(var + 1e-5)
    t = be_ref[...] - mean * s
    return s, t


def _amm_body(y_ref, st_ref, g_ref, be_ref, w_ref, o_ref):
    s, t = _affine(st_ref, g_ref, be_ref)
    z = y_ref[...] * s + t
    o_ref[...] = jnp.dot(z, w_ref[...], preferred_element_type=jnp.float32)


def _amm(y, st, g, be, w):
    return pl.pallas_call(
        _amm_body,
        grid=(N // BM,),
        in_specs=[
            pl.BlockSpec((BM, H), lambda i: (i, 0)),
            pl.BlockSpec((2, H), lambda i: (0, 0)),
            pl.BlockSpec((1, H), lambda i: (0, 0)),
            pl.BlockSpec((1, H), lambda i: (0, 0)),
            pl.BlockSpec((H, H), lambda i: (0, 0)),
        ],
        out_specs=pl.BlockSpec((BM, H), lambda i: (i, 0)),
        out_shape=jax.ShapeDtypeStruct((N, H), jnp.float32),
    )(y, st, g, be, w)


def _pool_body(y_ref, st_ref, g_ref, be_ref, batch_ref, wc_ref, bc_ref,
               o_ref, pacc_ref):
    i = pl.program_id(0)
    s, t = _affine(st_ref, g_ref, be_ref)
    z = y_ref[...] * s + t
    bidx = batch_ref[...]
    rows = []
    for gg in range(16):
        zg = jnp.where(bidx == gg, z, -jnp.inf)
        rows.append(jnp.max(zg, axis=0))
    cur = jnp.stack(rows, axis=0)

    @pl.when(i == 0)
    def _():
        pacc_ref[...] = cur

    @pl.when(i > 0)
    def _():
        pacc_ref[...] = jnp.maximum(pacc_ref[...], cur)

    @pl.when(i == N // BM - 1)
    def _():
        o_ref[...] = jnp.dot(pacc_ref[...], wc_ref[...],
                             preferred_element_type=jnp.float32) + bc_ref[...]


def _pool(y, st, g, be, batch2d, wc, bc):
    return pl.pallas_call(
        _pool_body,
        grid=(N // BM,),
        in_specs=[
            pl.BlockSpec((BM, H), lambda i: (i, 0)),
            pl.BlockSpec((2, H), lambda i: (0, 0)),
            pl.BlockSpec((1, H), lambda i: (0, 0)),
            pl.BlockSpec((1, H), lambda i: (0, 0)),
            pl.BlockSpec((BM, 1), lambda i: (i, 0)),
            pl.BlockSpec((H, H), lambda i: (0, 0)),
            pl.BlockSpec((1, H), lambda i: (0, 0)),
        ],
        out_specs=pl.BlockSpec((16, H), lambda i: (0, 0)),
        out_shape=jax.ShapeDtypeStruct((16, H), jnp.float32),
        scratch_shapes=[pltpu.VMEM((16, H), jnp.float32)],
    )(y, st, g, be, batch2d, wc, bc)


def kernel(x, edge_index, edge_attr, batch, Wl0, Wr0, att0, We0, b0, g0, be0,
           Wl1, att1, We1, b1, g1, be1, Wl2, att2, We2, b2, g2, be2, Wc, bc):
    src = edge_index[0]
    dst = edge_index[1]
    ea_flat = edge_attr.reshape(-1)

    xl, xr = _mm2(x, Wl0, Wr0)
    msg, den = _SC_GAT(xl, xr, src, dst, ea_flat, We0, att0)
    y, st = _post(msg, _dred(den), b0.reshape(1, -1))

    xl1 = _amm(y, st, g0.reshape(1, -1), be0.reshape(1, -1), Wl1)
    msg, den = _SC_GAT(xl1, xl1, src, dst, ea_flat, We1, att1)
    y, st = _post(msg, _dred(den), b1.reshape(1, -1))

    xl2 = _amm(y, st, g1.reshape(1, -1), be1.reshape(1, -1), Wl2)
    msg, den = _SC_GAT(xl2, xl2, src, dst, ea_flat, We2, att2)
    y, st = _post(msg, _dred(den), b2.reshape(1, -1))

    out = _pool(y, st, g2.reshape(1, -1), be2.reshape(1, -1),
                batch.reshape(-1, 1),
                jnp.pad(Wc, ((0, 0), (0, H - Wc.shape[1]))),
                jnp.pad(bc, (0, H - bc.shape[0])).reshape(1, -1))
    return out[:, :bc.shape[0]]


# speedup vs baseline: 1.0663x; 1.0663x over previous
import functools

import jax
import jax.numpy as jnp
from jax import lax
from jax.experimental import pallas as pl
from jax.experimental.pallas import tpu as pltpu
from jax.experimental.pallas import tpu_sc as plsc

N = 10000
E = 320000
H = 128
C = 40
NC = 2
NS = 16
NW = NC * NS
BM = 400


EPW = E // NW
CPW = EPW // C
SBC = 50
SBE = SBC * C
NSB = CPW // SBC
DEN = N + 16


def _build_sc_gat():
    mesh = plsc.VectorSubcoreMesh(core_axis_name="c", subcore_axis_name="s")
    CH = C // 2

    @functools.partial(
        pl.kernel,
        mesh=mesh,
        out_type=[
            jax.ShapeDtypeStruct((NC, N, H), jnp.float32),
            jax.ShapeDtypeStruct((NC, NS, DEN), jnp.float32),
        ],
        scratch_types=[
            pltpu.VMEM((C, H), jnp.float32),
            pltpu.VMEM((C, H), jnp.float32),
            pltpu.VMEM((C, H), jnp.float32),
            pltpu.VMEM((C, H), jnp.float32),
            pltpu.VMEM((CH, H), jnp.float32),
            pltpu.VMEM((CH, H), jnp.float32),
            pltpu.VMEM((CH,), jnp.int32),
            pltpu.VMEM((CH,), jnp.int32),
            pltpu.VMEM((SBE + 16,), jnp.int32),
            pltpu.VMEM((SBE + 16,), jnp.int32),
            pltpu.VMEM((SBE * 4 + 16,), jnp.float32),
            pltpu.VMEM((4, H), jnp.float32),
            pltpu.VMEM((H,), jnp.float32),
            pltpu.VMEM((DEN,), jnp.float32),
            pltpu.VMEM_SHARED((N, H), jnp.float32),
            pltpu.SemaphoreType.DMA,
            pltpu.SemaphoreType.DMA,
            pltpu.SemaphoreType.DMA,
            pltpu.SemaphoreType.DMA,
            pltpu.SemaphoreType.DMA,
            pltpu.SemaphoreType.DMA,
        ],
    )
    def sc_gat(xl_hbm, xr_hbm, src_hbm, dst_hbm, ea_hbm, we_hbm, att_hbm,
               out_hbm, den_hbm, xs0, xs1, xd0, xd1, mr0, mr1, dstsc0,
               dstsc1, srcbf, dstbf, eab, wev, attv, den_local, acc,
               sems0, sems1, semd0, semd1, semsc0, semsc1):
        cid = lax.axis_index("c")
        sid = lax.axis_index("s")
        wid = sid * NC + cid
        xsb = (xs0, xs1)
        xdb = (xd0, xd1)
        gsem = ((sems0, semd0), (sems1, semd1))
        mrh = (mr0, mr1)
        dsch = (dstsc0, dstsc1)
        scsem = (semsc0, semsc1)

        pltpu.sync_copy(we_hbm, wev)
        pltpu.sync_copy(att_hbm, attv)

        z16 = jnp.zeros((16,), jnp.float32)

        def zrow(r, carry):
            for j in range(H // 16):
                xs0[r, pl.ds(16 * j, 16)] = z16
            return carry

        lax.fori_loop(0, C, zrow, 0)

        def zden(r, carry):
            den_local[pl.ds(16 * r, 16)] = z16
            return carry

        lax.fori_loop(0, DEN // 16, zden, 0)

        row0 = sid * 632
        nfull = jnp.where(sid < NS - 1, 15, 13)

        def zacc(kk, carry):
            pltpu.sync_copy(xs0, acc.at[pl.ds(row0 + kk * C, C)])
            return carry

        lax.fori_loop(0, nfull, zacc, 0)

        @pl.when(sid < NS - 1)
        def _():
            pltpu.sync_copy(xs0.at[pl.ds(0, 32)],
                            acc.at[pl.ds(row0 + 600, 32)])

        plsc.subcore_barrier()

        wej = [[wev[d, pl.ds(16 * j, 16)] for j in range(8)] for d in range(4)]
        attj = [attv[pl.ds(16 * j, 16)] for j in range(8)]
        lane = lax.iota(jnp.int32, 16)
        bfly = [lane ^ sh for sh in (8, 4, 2, 1)]
        lane0 = lane == 0
        e0 = wid * EPW

        def issue(c, b):
            pltpu.async_copy(xl_hbm.at[srcbf.at[pl.ds(c * C, C)]],
                             xsb[b], gsem[b][0])
            pltpu.async_copy(xr_hbm.at[dstbf.at[pl.ds(c * C, C)]],
                             xdb[b], gsem[b][1])

        def wait_g(b):
            pltpu.make_async_copy(xl_hbm.at[pl.ds(0, C)], xsb[b],
                                  gsem[b][0]).wait()
            pltpu.make_async_copy(xr_hbm.at[pl.ds(0, C)], xdb[b],
                                  gsem[b][1]).wait()

        def wait_sc(hh):
            pltpu.make_async_copy(mrh[hh], acc.at[dsch[hh]],
                                  scsem[hh]).wait()

        def sb_body(sb, carry):
            sbe0 = e0 + sb * SBE
            pltpu.sync_copy(src_hbm.at[pl.ds(sbe0, SBE)],
                            srcbf.at[pl.ds(0, SBE)])
            pltpu.sync_copy(dst_hbm.at[pl.ds(sbe0, SBE)],
                            dstbf.at[pl.ds(0, SBE)])
            pltpu.sync_copy(ea_hbm.at[pl.ds(sbe0 * 4, SBE * 4)],
                            eab.at[pl.ds(0, SBE * 4)])
            issue(0, 0)
            issue(1, 1)

            def half(c, b, hh, first_sb):
                xs = xsb[b]
                xd = xdb[b]
                mr = mrh[hh]
                ofs = CH * hh

                @pl.when(jnp.logical_or(c >= 1, first_sb == 0))
                def _():
                    wait_sc(hh)

                def edge_body(e, ecarry):
                    for u in range(4):
                        k = ofs + 4 * e + u
                        eav = eab[pl.ds((c * C + k) * 4, 16)]
                        terms = []
                        for j in range(8):
                            q = (xs[k, pl.ds(16 * j, 16)]
                                 + xd[k, pl.ds(16 * j, 16)]
                                 + eav[0] * wej[0][j] + eav[1] * wej[1][j]
                                 + eav[2] * wej[2][j] + eav[3] * wej[3][j])
                            lr = jnp.maximum(q, 0.2 * q)
                            terms.append(attj[j] * lr)
                        while len(terms) > 1:
                            terms = [a + b for a, b in
                                     zip(terms[::2], terms[1::2])]
                        lacc = terms[0]
                        for bidx in bfly:
                            lacc = lacc + lax.gather(
                                lacc, bidx[:, None],
                                lax.GatherDimensionNumbers(
                                    offset_dims=(), collapsed_slice_dims=(0,),
                                    start_index_map=(0,)),
                                slice_sizes=(1,),
                                mode=lax.GatherScatterMode.PROMISE_IN_BOUNDS)
                        p = jnp.exp(lacc)
                        for j in range(8):
                            mr[4 * e + u, pl.ds(16 * j, 16)] = (
                                p * xs[k, pl.ds(16 * j, 16)])
                        dw = dstbf[pl.ds(c * C + k, 16)]
                        d_e = dw[0]
                        w = den_local[pl.ds(d_e, 16)]
                        den_local[pl.ds(d_e, 16)] = (
                            w + jnp.where(lane0, p, 0.0))
                    return ecarry

                lax.fori_loop(0, CH // 4, edge_body, 0)
                dsc = dsch[hh]
                dsc[pl.ds(0, 16)] = dstbf[pl.ds(c * C + ofs, 16)]
                dsc[pl.ds(4, 16)] = dstbf[pl.ds(c * C + ofs + 4, 16)]
                pltpu.async_copy(mr, acc.at[dsc], scsem[hh], add=True)

            def do_chunk(c, b, first_sb):
                wait_g(b)
                half(c, b, 0, first_sb)
                half(c, b, 1, first_sb)

                @pl.when(c + 2 < SBC)
                def _():
                    issue(c + 2, b)

            def pair_body(tt, carry2):
                do_chunk(2 * tt, 0, carry2)
                do_chunk(2 * tt + 1, 1, carry2)
                return carry2

            first = jnp.where(sb == 0, 1, 0)
            lax.fori_loop(0, SBC // 2, pair_body, first)
            return carry

        lax.fori_loop(0, NSB, sb_body, 0)
        wait_sc(0)
        wait_sc(1)
        pltpu.sync_copy(den_local, den_hbm.at[cid, sid])
        plsc.subcore_barrier()

        @pl.when(sid < NS - 1)
        def _():
            pltpu.sync_copy(acc.at[pl.ds(row0, 632)],
                            out_hbm.at[cid, pl.ds(row0, 632)])

        @pl.when(sid == NS - 1)
        def _():
            pltpu.sync_copy(acc.at[pl.ds(row0, 520)],
                            out_hbm.at[cid, pl.ds(row0, 520)])

    return sc_gat


_SC_GAT = _build_sc_gat()


def _mm2_body(x_ref, wl_ref, wr_ref, o1_ref, o2_ref):
    xb = x_ref[...]
    o1_ref[...] = jnp.dot(xb, wl_ref[...], preferred_element_type=jnp.float32)
    o2_ref[...] = jnp.dot(xb, wr_ref[...], preferred_element_type=jnp.float32)


def _mm2(x, wl, wr):
    m, k = x.shape
    n = wl.shape[1]
    return pl.pallas_call(
        _mm2_body,
        grid=(m // BM,),
        in_specs=[
            pl.BlockSpec((BM, k), lambda i: (i, 0)),
            pl.BlockSpec((k, n), lambda i: (0, 0)),
            pl.BlockSpec((k, n), lambda i: (0, 0)),
        ],
        out_specs=[
            pl.BlockSpec((BM, n), lambda i: (i, 0)),
            pl.BlockSpec((BM, n), lambda i: (i, 0)),
        ],
        out_shape=[
            jax.ShapeDtypeStruct((m, n), jnp.float32),
            jax.ShapeDtypeStruct((m, n), jnp.float32),
        ],
    )(x, wl, wr)


def _dred_body(d_ref, o_ref):
    o_ref[...] = jnp.sum(d_ref[...], axis=0, keepdims=True)


def _dred(den):
    d2 = den.reshape(NC * NS, DEN)
    out = pl.pallas_call(
        _dred_body,
        grid=(1,),
        in_specs=[pl.BlockSpec((NC * NS, DEN), lambda i: (0, 0))],
        out_specs=pl.BlockSpec((1, DEN), lambda i: (0, 0)),
        out_shape=jax.ShapeDtypeStruct((1, DEN), jnp.float32),
    )(d2)
    return out[:, :N].reshape(N, 1)


def _post_body(msg_ref, den_ref, b_ref, y_ref, st_ref):
    i = pl.program_id(0)
    num = msg_ref[0] + msg_ref[1]
    den = den_ref[...]
    y = jnp.maximum(num / (den + 1e-16) + b_ref[...], 0.0)
    y_ref[...] = y
    s0 = jnp.sum(y, axis=0, keepdims=True)
    s1 = jnp.sum(y * y, axis=0, keepdims=True)
    st = jnp.concatenate([s0, s1], axis=0)

    @pl.when(i == 0)
    def _():
        st_ref[...] = st

    @pl.when(i > 0)
    def _():
        st_ref[...] = st_ref[...] + st


def _post(msg, den2d, b):
    return pl.pallas_call(
        _post_body,
        grid=(N // BM,),
        in_specs=[
            pl.BlockSpec((NC, BM, H), lambda i: (0, i, 0)),
            pl.BlockSpec((BM, 1), lambda i: (i, 0)),
            pl.BlockSpec((1, H), lambda i: (0, 0)),
        ],
        out_specs=[
            pl.BlockSpec((BM, H), lambda i: (i, 0)),
            pl.BlockSpec((2, H), lambda i: (0, 0)),
        ],
        out_shape=[
            jax.ShapeDtypeStruct((N, H), jnp.float32),
            jax.ShapeDtypeStruct((2, H), jnp.float32),
        ],
    )(msg, den2d, b)


def _affine(st_ref, g_ref, be_ref):
    mean = st_ref[0:1, :] * (1.0 / N)
    var = st_ref[1:2, :] * (1.0 / N) - mean * mean
    s = g_ref[...] * lax.rsqrt(var + 1e-5)
    t = be_ref[...] - mean * s
    return s, t


def _amm_body(y_ref, st_ref, g_ref, be_ref, w_ref, o_ref):
    s, t = _affine(st_ref, g_ref, be_ref)
    z = y_ref[...] * s + t
    o_ref[...] = jnp.dot(z, w_ref[...], preferred_element_type=jnp.float32)


def _amm(y, st, g, be, w):
    return pl.pallas_call(
        _amm_body,
        grid=(N // BM,),
        in_specs=[
            pl.BlockSpec((BM, H), lambda i: (i, 0)),
            pl.BlockSpec((2, H), lambda i: (0, 0)),
            pl.BlockSpec((1, H), lambda i: (0, 0)),
            pl.BlockSpec((1, H), lambda i: (0, 0)),
            pl.BlockSpec((H, H), lambda i: (0, 0)),
        ],
        out_specs=pl.BlockSpec((BM, H), lambda i: (i, 0)),
        out_shape=jax.ShapeDtypeStruct((N, H), jnp.float32),
    )(y, st, g, be, w)


def _pool_body(y_ref, st_ref, g_ref, be_ref, batch_ref, wc_ref, bc_ref,
               o_ref, pacc_ref):
    i = pl.program_id(0)
    s, t = _affine(st_ref, g_ref, be_ref)
    z = y_ref[...] * s + t
    bidx = batch_ref[...]
    rows = []
    for gg in range(16):
        zg = jnp.where(bidx == gg, z, -jnp.inf)
        rows.append(jnp.max(zg, axis=0))
    cur = jnp.stack(rows, axis=0)

    @pl.when(i == 0)
    def _():
        pacc_ref[...] = cur

    @pl.when(i > 0)
    def _():
        pacc_ref[...] = jnp.maximum(pacc_ref[...], cur)

    @pl.when(i == N // BM - 1)
    def _():
        o_ref[...] = jnp.dot(pacc_ref[...], wc_ref[...],
                             preferred_element_type=jnp.float32) + bc_ref[...]


def _pool(y, st, g, be, batch2d, wc, bc):
    return pl.pallas_call(
        _pool_body,
        grid=(N // BM,),
        in_specs=[
            pl.BlockSpec((BM, H), lambda i: (i, 0)),
            pl.BlockSpec((2, H), lambda i: (0, 0)),
            pl.BlockSpec((1, H), lambda i: (0, 0)),
            pl.BlockSpec((1, H), lambda i: (0, 0)),
            pl.BlockSpec((BM, 1), lambda i: (i, 0)),
            pl.BlockSpec((H, H), lambda i: (0, 0)),
            pl.BlockSpec((1, H), lambda i: (0, 0)),
        ],
        out_specs=pl.BlockSpec((16, H), lambda i: (0, 0)),
        out_shape=jax.ShapeDtypeStruct((16, H), jnp.float32),
        scratch_shapes=[pltpu.VMEM((16, H), jnp.float32)],
    )(y, st, g, be, batch2d, wc, bc)


def kernel(x, edge_index, edge_attr, batch, Wl0, Wr0, att0, We0, b0, g0, be0,
           Wl1, att1, We1, b1, g1, be1, Wl2, att2, We2, b2, g2, be2, Wc, bc):
    src = edge_index[0]
    dst = edge_index[1]
    ea_flat = edge_attr.reshape(-1)

    xl, xr = _mm2(x, Wl0, Wr0)
    msg, den = _SC_GAT(xl, xr, src, dst, ea_flat, We0, att0)
    y, st = _post(msg, _dred(den), b0.reshape(1, -1))

    xl1 = _amm(y, st, g0.reshape(1, -1), be0.reshape(1, -1), Wl1)
    msg, den = _SC_GAT(xl1, xl1, src, dst, ea_flat, We1, att1)
    y, st = _post(msg, _dred(den), b1.reshape(1, -1))

    xl2 = _amm(y, st, g1.reshape(1, -1), be1.reshape(1, -1), Wl2)
    msg, den = _SC_GAT(xl2, xl2, src, dst, ea_flat, We2, att2)
    y, st = _post(msg, _dred(den), b2.reshape(1, -1))

    out = _pool(y, st, g2.reshape(1, -1), be2.reshape(1, -1),
                batch.reshape(-1, 1),
                jnp.pad(Wc, ((0, 0), (0, H - Wc.shape[1]))),
                jnp.pad(bc, (0, H - bc.shape[0])).reshape(1, -1))
    return out[:, :bc.shape[0]]
